# baseline (speedup 1.0000x reference)
import functools

import jax
import jax.numpy as jnp
import numpy as np
from jax.experimental import pallas as pl
from jax.experimental.pallas import tpu as pltpu
from jax.experimental.pallas import tpu_sc as plsc

_V, _D, _H, _T = 1000000, 64, 64, 50
_NH = 4
_DH = _H // _NH
_L = 20

_GROUP_BATCH = 4
_GR = _GROUP_BATCH * _L
_GROUPS_PER_BLOCK = 8
_BLOCK_ROWS = _GR * _GROUPS_PER_BLOCK

_NW = 32
_WIN = 128
_RING = 4


def _sc_gather(emb_pad, idx2d):
    n_wins = idx2d.shape[0]
    d = emb_pad.shape[1]
    wins_per_worker = n_wins // _NW
    mesh = plsc.VectorSubcoreMesh(core_axis_name="c", subcore_axis_name="s")

    @functools.partial(
        pl.kernel,
        out_type=jax.ShapeDtypeStruct((n_wins * _WIN, d), emb_pad.dtype),
        mesh=mesh,
        scratch_types=[
            pltpu.VMEM((wins_per_worker, _WIN), jnp.int32),
            pltpu.VMEM((_RING * _WIN, d), emb_pad.dtype),
        ] + [pltpu.SemaphoreType.DMA] * _RING,
    )
    def gather_kernel(emb_hbm, i_hbm, o_hbm, idx_v, rows_v, *sems):
        wid = jax.lax.axis_index("s") * 2 + jax.lax.axis_index("c")
        win0 = wid * wins_per_worker

        pltpu.sync_copy(i_hbm.at[pl.ds(win0, wins_per_worker)], idx_v)

        def fire(slot, w):
            pltpu.async_copy(
                emb_hbm.at[idx_v.at[w]],
                rows_v.at[pl.ds(slot * _WIN, _WIN)],
                sems[slot],
            )

        def drain(slot):
            pltpu.make_async_copy(
                emb_hbm.at[idx_v.at[0]],
                rows_v.at[pl.ds(slot * _WIN, _WIN)],
                sems[slot],
            ).wait()

        for j in range(_RING):
            fire(j, j)

        @pl.loop(0, wins_per_worker // _RING)
        def _(c):
            for j in range(_RING):
                w = c * _RING + j
                drain(j)
                pltpu.sync_copy(
                    rows_v.at[pl.ds(j * _WIN, _WIN)],
                    o_hbm.at[pl.ds((win0 + w) * _WIN, _WIN)],
                )

                @pl.when(c < wins_per_worker // _RING - 1)
                def _():
                    fire(j, w + _RING)

    return gather_kernel(emb_pad, idx2d)


def _bdot(a, b):
    return jnp.dot(a.astype(jnp.bfloat16), b,
                   preferred_element_type=jnp.float32)


def _dense_body(h_ref, wqkv_ref, wo_ref, w1_ref, b1_ref, w2_ref, b2_ref,
                whead_ref, bhead_ref, bias_ref,
                zmu_ref, zsd_ref, smu_ref, ssd_ref):
    wqkv = wqkv_ref[...]
    wo = wo_ref[...]
    w1 = w1_ref[...]
    b1 = b1_ref[...]
    w2 = w2_ref[...]
    b2 = b2_ref[...]
    whead = whead_ref[...]
    bhead = bhead_ref[...]
    bias = bias_ref[...]

    h = h_ref[:, 0:_H]
    qkv = _bdot(h, wqkv)

    head_id = jax.lax.broadcasted_iota(jnp.int32, (_GR, _H), 1) // _DH
    zero = jnp.zeros((), jnp.bfloat16)
    m2 = jnp.concatenate(
        [(head_id == m).astype(jnp.bfloat16) for m in range(_NH)], axis=0)

    outs = []
    for g in range(_GROUPS_PER_BLOCK):
        r0 = g * _GR
        q = qkv[r0:r0 + _GR, 0:_H].astype(jnp.bfloat16)
        k = qkv[r0:r0 + _GR, _H:2 * _H].astype(jnp.bfloat16)
        v = qkv[r0:r0 + _GR, 2 * _H:3 * _H].astype(jnp.bfloat16)

        k2 = jnp.concatenate(
            [jnp.where(head_id == m, k, zero) for m in range(_NH)], axis=0)
        v2 = jnp.concatenate(
            [jnp.where(head_id == m, v, zero) for m in range(_NH)], axis=0)
        v3 = jnp.concatenate([v2, m2], axis=1)

        s = jax.lax.dot_general(
            q, k2, (((1,), (1,)), ((), ())),
            preferred_element_type=jnp.float32)
        p = jnp.exp(s + bias)

        c = _bdot(p, v3)
        outs.append(c[:, 0:_H] / c[:, _H:2 * _H])

    o = jnp.concatenate(outs, axis=0)
    h = h + _bdot(o, wo)
    m = jnp.maximum(_bdot(h, w1) + b1, 0.0)
    m = jnp.maximum(_bdot(m, w2) + b2, 0.0)
    h = h + m

    hd = _bdot(h, whead) + bhead
    gb = _BLOCK_ROWS // _L
    zmu_ref[...] = hd[:, 0:_T].reshape(gb, _L, _T)
    zsd_ref[...] = jnp.exp(hd[:, _T:2 * _T]).reshape(gb, _L, _T)
    smu_ref[...] = hd[:, 2 * _T:2 * _T + 1].reshape(gb, _L, 1)
    ssd_ref[...] = jnp.exp(hd[:, 2 * _T + 1:2 * _T + 2]).reshape(gb, _L, 1)


def _dense_stage(h_flat, wqkv, wo, w1, b1, w2, b2, whead, bhead, bias):
    n = h_flat.shape[0]
    grid = (n // _BLOCK_ROWS,)
    const = lambda shape: pl.BlockSpec(shape, lambda i: (0, 0))
    return pl.pallas_call(
        _dense_body,
        grid=grid,
        in_specs=[
            pl.BlockSpec((_BLOCK_ROWS, 2 * _H), lambda i: (i, 0)),
            const(wqkv.shape),
            const(wo.shape),
            const(w1.shape),
            const(b1.shape),
            const(w2.shape),
            const(b2.shape),
            const(whead.shape),
            const(bhead.shape),
            const(bias.shape),
        ],
        out_specs=[
            pl.BlockSpec((_BLOCK_ROWS // _L, _L, _T), lambda i: (i, 0, 0)),
            pl.BlockSpec((_BLOCK_ROWS // _L, _L, _T), lambda i: (i, 0, 0)),
            pl.BlockSpec((_BLOCK_ROWS // _L, _L, 1), lambda i: (i, 0, 0)),
            pl.BlockSpec((_BLOCK_ROWS // _L, _L, 1), lambda i: (i, 0, 0)),
        ],
        out_shape=[
            jax.ShapeDtypeStruct((n // _L, _L, _T), jnp.float32),
            jax.ShapeDtypeStruct((n // _L, _L, _T), jnp.float32),
            jax.ShapeDtypeStruct((n // _L, _L, 1), jnp.float32),
            jax.ShapeDtypeStruct((n // _L, _L, 1), jnp.float32),
        ],
        compiler_params=pltpu.CompilerParams(
            dimension_semantics=("parallel",),
        ),
    )(h_flat, wqkv, wo, w1, b1, w2, b2, whead, bhead, bias)


def _cross_batch_bias():
    ri = np.arange(_GR)[:, None] // _L
    cj = (np.arange(_NH * _GR)[None, :] % _GR) // _L
    return np.where(ri == cj, 0.0, -1e30).astype(np.float32)


def kernel(x, emb, attn_w, mlp_w, mlp_b, zmu_w, zmu_b, zlv_w, zlv_b,
           smu_w, smu_b, slv_w, slv_b):
    b, l = x.shape
    n = b * l
    emb_pad = jnp.pad(emb, ((0, 0), (0, 2 * _H - emb.shape[1])))
    idx2d = x.reshape(n // _WIN, _WIN).astype(jnp.int32)
    h_flat = _sc_gather(emb_pad, idx2d)

    isq = 1.0 / np.sqrt(_DH)
    wqkv = jnp.concatenate(
        [attn_w[0, 0] * isq, attn_w[0, 1], attn_w[0, 2]],
        axis=1).astype(jnp.bfloat16)
    wo = attn_w[0, 3].astype(jnp.bfloat16)
    w1, w2 = mlp_w[0, 0].astype(jnp.bfloat16), mlp_w[0, 1].astype(jnp.bfloat16)
    b1, b2 = mlp_b[0, 0].reshape(1, _H), mlp_b[0, 1].reshape(1, _H)
    whead = jnp.concatenate(
        [zmu_w, 0.5 * zlv_w, smu_w, 0.5 * slv_w], axis=1).astype(jnp.bfloat16)
    bhead = jnp.concatenate(
        [zmu_b, 0.5 * zlv_b, smu_b, 0.5 * slv_b]).reshape(1, 2 * _T + 2)
    bias = jnp.asarray(_cross_batch_bias())

    zmu, zsd, smu, ssd = _dense_stage(
        h_flat, wqkv, wo, w1, b1, w2, b2, whead, bhead, bias)

    return (zmu.reshape(b, l, _T), zsd.reshape(b, l, _T),
            smu.reshape(b, l, 1), ssd.reshape(b, l, 1))


# speedup vs baseline: 1.0802x; 1.0802x over previous
import functools

import jax
import jax.numpy as jnp
import numpy as np
from jax.experimental import pallas as pl
from jax.experimental.pallas import tpu as pltpu
from jax.experimental.pallas import tpu_sc as plsc

_V, _D, _H, _T = 1000000, 64, 64, 50
_NH = 4
_DH = _H // _NH
_L = 20

_GROUP_BATCH = 8
_GR = _GROUP_BATCH * _L
_GROUPS_PER_BLOCK = 4
_BLOCK_ROWS = _GR * _GROUPS_PER_BLOCK

_NW = 32
_WIN = 128
_RING = 4


def _sc_gather(emb_pad, idx2d):
    n_wins = idx2d.shape[0]
    d = emb_pad.shape[1]
    wins_per_worker = n_wins // _NW
    mesh = plsc.VectorSubcoreMesh(core_axis_name="c", subcore_axis_name="s")

    @functools.partial(
        pl.kernel,
        out_type=jax.ShapeDtypeStruct((n_wins * _WIN, d), emb_pad.dtype),
        mesh=mesh,
        scratch_types=[
            pltpu.VMEM((wins_per_worker, _WIN), jnp.int32),
            pltpu.VMEM((_RING * _WIN, d), emb_pad.dtype),
        ] + [pltpu.SemaphoreType.DMA] * _RING,
    )
    def gather_kernel(emb_hbm, i_hbm, o_hbm, idx_v, rows_v, *sems):
        wid = jax.lax.axis_index("s") * 2 + jax.lax.axis_index("c")
        win0 = wid * wins_per_worker

        pltpu.sync_copy(i_hbm.at[pl.ds(win0, wins_per_worker)], idx_v)

        def fire(slot, w):
            pltpu.async_copy(
                emb_hbm.at[idx_v.at[w]],
                rows_v.at[pl.ds(slot * _WIN, _WIN)],
                sems[slot],
            )

        def drain(slot):
            pltpu.make_async_copy(
                emb_hbm.at[idx_v.at[0]],
                rows_v.at[pl.ds(slot * _WIN, _WIN)],
                sems[slot],
            ).wait()

        for j in range(_RING):
            fire(j, j)

        @pl.loop(0, wins_per_worker // _RING)
        def _(c):
            for j in range(_RING):
                w = c * _RING + j
                drain(j)
                pltpu.sync_copy(
                    rows_v.at[pl.ds(j * _WIN, _WIN)],
                    o_hbm.at[pl.ds((win0 + w) * _WIN, _WIN)],
                )

                @pl.when(c < wins_per_worker // _RING - 1)
                def _():
                    fire(j, w + _RING)

    return gather_kernel(emb_pad, idx2d)


def _bdot(a, b):
    return jnp.dot(a.astype(jnp.bfloat16), b,
                   preferred_element_type=jnp.float32)


def _dense_body(h_ref, wqkv_ref, wo_ref, w1_ref, b1_ref, w2_ref, b2_ref,
                whead_ref, bhead_ref, bias_ref,
                zmu_ref, zsd_ref, smu_ref, ssd_ref):
    wqkv = wqkv_ref[...]
    wo = wo_ref[...]
    w1 = w1_ref[...]
    b1 = b1_ref[...]
    w2 = w2_ref[...]
    b2 = b2_ref[...]
    whead = whead_ref[...]
    bhead = bhead_ref[...]
    bias = bias_ref[...]

    h = h_ref[:, 0:_H]
    qkv = _bdot(h, wqkv)

    head_id = jax.lax.broadcasted_iota(jnp.int32, (_GR, _H), 1) // _DH
    zero = jnp.zeros((), jnp.bfloat16)
    m2 = jnp.concatenate(
        [(head_id == m).astype(jnp.bfloat16) for m in range(_NH)], axis=0)

    outs = []
    for g in range(_GROUPS_PER_BLOCK):
        r0 = g * _GR
        q = qkv[r0:r0 + _GR, 0:_H].astype(jnp.bfloat16)
        k = qkv[r0:r0 + _GR, _H:2 * _H].astype(jnp.bfloat16)
        v = qkv[r0:r0 + _GR, 2 * _H:3 * _H].astype(jnp.bfloat16)

        k2 = jnp.concatenate(
            [jnp.where(head_id == m, k, zero) for m in range(_NH)], axis=0)
        v2 = jnp.concatenate(
            [jnp.where(head_id == m, v, zero) for m in range(_NH)], axis=0)
        v3 = jnp.concatenate([v2, m2], axis=1)

        s = jax.lax.dot_general(
            q, k2, (((1,), (1,)), ((), ())),
            preferred_element_type=jnp.float32)
        p = jnp.exp(s + bias)

        c = _bdot(p, v3)
        outs.append(c[:, 0:_H] / c[:, _H:2 * _H])

    o = jnp.concatenate(outs, axis=0)
    h = h + _bdot(o, wo)
    m = jnp.maximum(_bdot(h, w1) + b1, 0.0)
    m = jnp.maximum(_bdot(m, w2) + b2, 0.0)
    h = h + m

    hd = _bdot(h, whead) + bhead
    gb = _BLOCK_ROWS // _L
    zmu_ref[...] = hd[:, 0:_T].reshape(gb, _L, _T)
    zsd_ref[...] = jnp.exp(hd[:, _T:2 * _T]).reshape(gb, _L, _T)
    smu_ref[...] = hd[:, 2 * _T:2 * _T + 1].reshape(gb, _L, 1)
    ssd_ref[...] = jnp.exp(hd[:, 2 * _T + 1:2 * _T + 2]).reshape(gb, _L, 1)


def _dense_stage(h_flat, wqkv, wo, w1, b1, w2, b2, whead, bhead, bias):
    n = h_flat.shape[0]
    grid = (n // _BLOCK_ROWS,)
    const = lambda shape: pl.BlockSpec(shape, lambda i: (0, 0))
    return pl.pallas_call(
        _dense_body,
        grid=grid,
        in_specs=[
            pl.BlockSpec((_BLOCK_ROWS, 2 * _H), lambda i: (i, 0)),
            const(wqkv.shape),
            const(wo.shape),
            const(w1.shape),
            const(b1.shape),
            const(w2.shape),
            const(b2.shape),
            const(whead.shape),
            const(bhead.shape),
            const(bias.shape),
        ],
        out_specs=[
            pl.BlockSpec((_BLOCK_ROWS // _L, _L, _T), lambda i: (i, 0, 0)),
            pl.BlockSpec((_BLOCK_ROWS // _L, _L, _T), lambda i: (i, 0, 0)),
            pl.BlockSpec((_BLOCK_ROWS // _L, _L, 1), lambda i: (i, 0, 0)),
            pl.BlockSpec((_BLOCK_ROWS // _L, _L, 1), lambda i: (i, 0, 0)),
        ],
        out_shape=[
            jax.ShapeDtypeStruct((n // _L, _L, _T), jnp.float32),
            jax.ShapeDtypeStruct((n // _L, _L, _T), jnp.float32),
            jax.ShapeDtypeStruct((n // _L, _L, 1), jnp.float32),
            jax.ShapeDtypeStruct((n // _L, _L, 1), jnp.float32),
        ],
        compiler_params=pltpu.CompilerParams(
            dimension_semantics=("parallel",),
        ),
    )(h_flat, wqkv, wo, w1, b1, w2, b2, whead, bhead, bias)


def _cross_batch_bias():
    ri = np.arange(_GR)[:, None] // _L
    cj = (np.arange(_NH * _GR)[None, :] % _GR) // _L
    return np.where(ri == cj, 0.0, -1e30).astype(np.float32)


def kernel(x, emb, attn_w, mlp_w, mlp_b, zmu_w, zmu_b, zlv_w, zlv_b,
           smu_w, smu_b, slv_w, slv_b):
    b, l = x.shape
    n = b * l
    emb_pad = jnp.pad(emb, ((0, 0), (0, 2 * _H - emb.shape[1])))
    idx2d = x.reshape(n // _WIN, _WIN).astype(jnp.int32)

    isq = 1.0 / np.sqrt(_DH)
    wqkv = jnp.concatenate(
        [attn_w[0, 0] * isq, attn_w[0, 1], attn_w[0, 2]],
        axis=1).astype(jnp.bfloat16)
    wo = attn_w[0, 3].astype(jnp.bfloat16)
    w1, w2 = mlp_w[0, 0].astype(jnp.bfloat16), mlp_w[0, 1].astype(jnp.bfloat16)
    b1, b2 = mlp_b[0, 0].reshape(1, _H), mlp_b[0, 1].reshape(1, _H)
    whead = jnp.concatenate(
        [zmu_w, 0.5 * zlv_w, smu_w, 0.5 * slv_w], axis=1).astype(jnp.bfloat16)
    bhead = jnp.concatenate(
        [zmu_b, 0.5 * zlv_b, smu_b, 0.5 * slv_b]).reshape(1, 2 * _T + 2)
    bias = jnp.asarray(_cross_batch_bias())

    half = idx2d.shape[0] // 2
    parts = []
    for idx_c in (idx2d[:half], idx2d[half:]):
        h_c = _sc_gather(emb_pad, idx_c)
        parts.append(_dense_stage(
            h_c, wqkv, wo, w1, b1, w2, b2, whead, bhead, bias))

    zmu, zsd, smu, ssd = (
        jnp.concatenate([p[i] for p in parts], axis=0) for i in range(4))
    return (zmu.reshape(b, l, _T), zsd.reshape(b, l, _T),
            smu.reshape(b, l, 1), ssd.reshape(b, l, 1))


# speedup vs baseline: 1.2167x; 1.1264x over previous
import functools

import jax
import jax.numpy as jnp
import numpy as np
from jax.experimental import pallas as pl
from jax.experimental.pallas import tpu as pltpu
from jax.experimental.pallas import tpu_sc as plsc

_V, _D, _H, _T = 1000000, 64, 64, 50
_NH = 4
_DH = _H // _NH
_L = 20

_GROUP_BATCH = 8
_GR = _GROUP_BATCH * _L
_GROUPS_PER_BLOCK = 8
_BLOCK_ROWS = _GR * _GROUPS_PER_BLOCK

_NW = 32
_WIN = 128
_RING = 4


def _sc_gather(emb_pad, idx2d):
    n_wins = idx2d.shape[0]
    d = emb_pad.shape[1]
    wins_per_worker = n_wins // _NW
    mesh = plsc.VectorSubcoreMesh(core_axis_name="c", subcore_axis_name="s")

    @functools.partial(
        pl.kernel,
        out_type=jax.ShapeDtypeStruct((n_wins * _WIN, d), emb_pad.dtype),
        mesh=mesh,
        scratch_types=[
            pltpu.VMEM((wins_per_worker, _WIN), jnp.int32),
            pltpu.VMEM((_RING * _WIN, d), emb_pad.dtype),
        ] + [pltpu.SemaphoreType.DMA] * _RING,
    )
    def gather_kernel(emb_hbm, i_hbm, o_hbm, idx_v, rows_v, *sems):
        wid = jax.lax.axis_index("s") * 2 + jax.lax.axis_index("c")
        win0 = wid * wins_per_worker

        pltpu.sync_copy(i_hbm.at[pl.ds(win0, wins_per_worker)], idx_v)

        def fire(slot, w):
            pltpu.async_copy(
                emb_hbm.at[idx_v.at[w]],
                rows_v.at[pl.ds(slot * _WIN, _WIN)],
                sems[slot],
            )

        def drain(slot):
            pltpu.make_async_copy(
                emb_hbm.at[idx_v.at[0]],
                rows_v.at[pl.ds(slot * _WIN, _WIN)],
                sems[slot],
            ).wait()

        for j in range(_RING):
            fire(j, j)

        @pl.loop(0, wins_per_worker // _RING)
        def _(c):
            for j in range(_RING):
                w = c * _RING + j
                drain(j)
                pltpu.sync_copy(
                    rows_v.at[pl.ds(j * _WIN, _WIN)],
                    o_hbm.at[pl.ds((win0 + w) * _WIN, _WIN)],
                )

                @pl.when(c < wins_per_worker // _RING - 1)
                def _():
                    fire(j, w + _RING)

    return gather_kernel(emb_pad, idx2d)


def _bdot(a, b):
    return jnp.dot(a.astype(jnp.bfloat16), b,
                   preferred_element_type=jnp.float32)


def _dense_body(h_ref, wqkv_ref, wo_ref, w1_ref, b1_ref, w2_ref, b2_ref,
                whead_ref, bhead_ref, bias_ref,
                zmu_ref, zsd_ref, smu_ref, ssd_ref):
    wqkv = wqkv_ref[...]
    wo = wo_ref[...]
    w1 = w1_ref[...]
    b1 = b1_ref[...]
    w2 = w2_ref[...]
    b2 = b2_ref[...]
    whead = whead_ref[...]
    bhead = bhead_ref[...]
    bias = bias_ref[...]

    h = h_ref[:, 0:_H]
    qkv = _bdot(h, wqkv)

    head_id = jax.lax.broadcasted_iota(jnp.int32, (_GR, _H), 1) // _DH
    zero = jnp.zeros((), jnp.bfloat16)
    m2 = jnp.concatenate(
        [(head_id == m).astype(jnp.bfloat16) for m in range(_NH)], axis=0)

    outs = []
    for g in range(_GROUPS_PER_BLOCK):
        r0 = g * _GR
        q = qkv[r0:r0 + _GR, 0:_H].astype(jnp.bfloat16)
        k = qkv[r0:r0 + _GR, _H:2 * _H].astype(jnp.bfloat16)
        v = qkv[r0:r0 + _GR, 2 * _H:3 * _H].astype(jnp.bfloat16)

        k2 = jnp.concatenate(
            [jnp.where(head_id == m, k, zero) for m in range(_NH)], axis=0)
        v2 = jnp.concatenate(
            [jnp.where(head_id == m, v, zero) for m in range(_NH)], axis=0)
        v3 = jnp.concatenate([v2, m2], axis=1)

        s = jax.lax.dot_general(
            q, k2, (((1,), (1,)), ((), ())),
            preferred_element_type=jnp.float32)
        p = jnp.exp(s + bias)

        c = _bdot(p, v3)
        outs.append(c[:, 0:_H] / c[:, _H:2 * _H])

    o = jnp.concatenate(outs, axis=0)
    h = h + _bdot(o, wo)
    m = jnp.maximum(_bdot(h, w1) + b1, 0.0)
    m = jnp.maximum(_bdot(m, w2) + b2, 0.0)
    h = h + m

    hd = _bdot(h, whead) + bhead
    gb = _BLOCK_ROWS // _L
    zmu_ref[...] = hd[:, 0:_T].reshape(gb, _L, _T)
    zsd_ref[...] = jnp.exp(hd[:, _T:2 * _T]).reshape(gb, _L, _T)
    smu_ref[...] = hd[:, 2 * _T:2 * _T + 1].reshape(gb, _L, 1)
    ssd_ref[...] = jnp.exp(hd[:, 2 * _T + 1:2 * _T + 2]).reshape(gb, _L, 1)


def _dense_stage(h_flat, wqkv, wo, w1, b1, w2, b2, whead, bhead, bias):
    n = h_flat.shape[0]
    grid = (n // _BLOCK_ROWS,)
    const = lambda shape: pl.BlockSpec(shape, lambda i: (0, 0))
    return pl.pallas_call(
        _dense_body,
        grid=grid,
        in_specs=[
            pl.BlockSpec((_BLOCK_ROWS, 2 * _H), lambda i: (i, 0)),
            const(wqkv.shape),
            const(wo.shape),
            const(w1.shape),
            const(b1.shape),
            const(w2.shape),
            const(b2.shape),
            const(whead.shape),
            const(bhead.shape),
            const(bias.shape),
        ],
        out_specs=[
            pl.BlockSpec((_BLOCK_ROWS // _L, _L, _T), lambda i: (i, 0, 0)),
            pl.BlockSpec((_BLOCK_ROWS // _L, _L, _T), lambda i: (i, 0, 0)),
            pl.BlockSpec((_BLOCK_ROWS // _L, _L, 1), lambda i: (i, 0, 0)),
            pl.BlockSpec((_BLOCK_ROWS // _L, _L, 1), lambda i: (i, 0, 0)),
        ],
        out_shape=[
            jax.ShapeDtypeStruct((n // _L, _L, _T), jnp.float32),
            jax.ShapeDtypeStruct((n // _L, _L, _T), jnp.float32),
            jax.ShapeDtypeStruct((n // _L, _L, 1), jnp.float32),
            jax.ShapeDtypeStruct((n // _L, _L, 1), jnp.float32),
        ],
        compiler_params=pltpu.CompilerParams(
            dimension_semantics=("parallel",),
        ),
    )(h_flat, wqkv, wo, w1, b1, w2, b2, whead, bhead, bias)


def _cross_batch_bias():
    ri = np.arange(_GR)[:, None] // _L
    cj = (np.arange(_NH * _GR)[None, :] % _GR) // _L
    return np.where(ri == cj, 0.0, -1e30).astype(np.float32)


def kernel(x, emb, attn_w, mlp_w, mlp_b, zmu_w, zmu_b, zlv_w, zlv_b,
           smu_w, smu_b, slv_w, slv_b):
    b, l = x.shape
    n = b * l
    emb_pad = jnp.pad(emb, ((0, 0), (0, 2 * _H - emb.shape[1])))
    idx2d = x.reshape(n // _WIN, _WIN).astype(jnp.int32)

    isq = 1.0 / np.sqrt(_DH)
    wqkv = jnp.concatenate(
        [attn_w[0, 0] * isq, attn_w[0, 1], attn_w[0, 2]],
        axis=1).astype(jnp.bfloat16)
    wo = attn_w[0, 3].astype(jnp.bfloat16)
    w1, w2 = mlp_w[0, 0].astype(jnp.bfloat16), mlp_w[0, 1].astype(jnp.bfloat16)
    b1, b2 = mlp_b[0, 0].reshape(1, _H), mlp_b[0, 1].reshape(1, _H)
    whead = jnp.concatenate(
        [zmu_w, 0.5 * zlv_w, smu_w, 0.5 * slv_w], axis=1).astype(jnp.bfloat16)
    bhead = jnp.concatenate(
        [zmu_b, 0.5 * zlv_b, smu_b, 0.5 * slv_b]).reshape(1, 2 * _T + 2)
    bias = jnp.asarray(_cross_batch_bias())

    h_flat = _sc_gather(emb_pad, idx2d)
    zmu, zsd, smu, ssd = _dense_stage(
        h_flat, wqkv, wo, w1, b1, w2, b2, whead, bhead, bias)
    return (zmu.reshape(b, l, _T), zsd.reshape(b, l, _T),
            smu.reshape(b, l, 1), ssd.reshape(b, l, 1))


# speedup vs baseline: 1.2967x; 1.0657x over previous
import functools

import jax
import jax.numpy as jnp
import numpy as np
from jax.experimental import pallas as pl
from jax.experimental.pallas import tpu as pltpu
from jax.experimental.pallas import tpu_sc as plsc

_V, _D, _H, _T = 1000000, 64, 64, 50
_NH = 4
_DH = _H // _NH
_L = 20

_GROUP_BATCH = 8
_GR = _GROUP_BATCH * _L
_GROUPS_PER_BLOCK = 16
_BLOCK_ROWS = _GR * _GROUPS_PER_BLOCK

_NW = 32
_WIN = 128
_RING = 4


def _sc_gather(emb_pad, idx2d):
    n_wins = idx2d.shape[0]
    d = emb_pad.shape[1]
    wins_per_worker = n_wins // _NW
    mesh = plsc.VectorSubcoreMesh(core_axis_name="c", subcore_axis_name="s")

    @functools.partial(
        pl.kernel,
        out_type=jax.ShapeDtypeStruct((n_wins * _WIN, d), emb_pad.dtype),
        mesh=mesh,
        scratch_types=[
            pltpu.VMEM((wins_per_worker, _WIN), jnp.int32),
            pltpu.VMEM((_RING * _WIN, d), emb_pad.dtype),
        ] + [pltpu.SemaphoreType.DMA] * _RING,
    )
    def gather_kernel(emb_hbm, i_hbm, o_hbm, idx_v, rows_v, *sems):
        wid = jax.lax.axis_index("s") * 2 + jax.lax.axis_index("c")
        win0 = wid * wins_per_worker

        pltpu.sync_copy(i_hbm.at[pl.ds(win0, wins_per_worker)], idx_v)

        def fire(slot, w):
            pltpu.async_copy(
                emb_hbm.at[idx_v.at[w]],
                rows_v.at[pl.ds(slot * _WIN, _WIN)],
                sems[slot],
            )

        def drain(slot):
            pltpu.make_async_copy(
                emb_hbm.at[idx_v.at[0]],
                rows_v.at[pl.ds(slot * _WIN, _WIN)],
                sems[slot],
            ).wait()

        for j in range(_RING):
            fire(j, j)

        @pl.loop(0, wins_per_worker // _RING)
        def _(c):
            for j in range(_RING):
                w = c * _RING + j
                drain(j)
                pltpu.sync_copy(
                    rows_v.at[pl.ds(j * _WIN, _WIN)],
                    o_hbm.at[pl.ds((win0 + w) * _WIN, _WIN)],
                )

                @pl.when(c < wins_per_worker // _RING - 1)
                def _():
                    fire(j, w + _RING)

    return gather_kernel(emb_pad, idx2d)


def _bdot(a, b):
    return jnp.dot(a.astype(jnp.bfloat16), b,
                   preferred_element_type=jnp.float32)


def _dense_body(h_ref, wqkv_ref, wo_ref, w1_ref, b1_ref, w2_ref, b2_ref,
                whead_ref, bhead_ref, bias_ref,
                zmu_ref, zsd_ref, smu_ref, ssd_ref):
    wqkv = wqkv_ref[...]
    wo = wo_ref[...]
    w1 = w1_ref[...]
    b1 = b1_ref[...]
    w2 = w2_ref[...]
    b2 = b2_ref[...]
    whead = whead_ref[...]
    bhead = bhead_ref[...]
    bias = bias_ref[...]

    h = h_ref[:, 0:_H]
    qkv = _bdot(h, wqkv)

    head_id = jax.lax.broadcasted_iota(jnp.int32, (_GR, _H), 1) // _DH
    zero = jnp.zeros((), jnp.bfloat16)
    m2 = jnp.concatenate(
        [(head_id == m).astype(jnp.bfloat16) for m in range(_NH)], axis=0)

    outs = []
    for g in range(_GROUPS_PER_BLOCK):
        r0 = g * _GR
        q = qkv[r0:r0 + _GR, 0:_H].astype(jnp.bfloat16)
        k = qkv[r0:r0 + _GR, _H:2 * _H].astype(jnp.bfloat16)
        v = qkv[r0:r0 + _GR, 2 * _H:3 * _H].astype(jnp.bfloat16)

        k2 = jnp.concatenate(
            [jnp.where(head_id == m, k, zero) for m in range(_NH)], axis=0)
        v2 = jnp.concatenate(
            [jnp.where(head_id == m, v, zero) for m in range(_NH)], axis=0)
        v3 = jnp.concatenate([v2, m2], axis=1)

        s = jax.lax.dot_general(
            q, k2, (((1,), (1,)), ((), ())),
            preferred_element_type=jnp.float32)
        p = jnp.exp(s + bias)

        c = _bdot(p, v3)
        outs.append(c[:, 0:_H] / c[:, _H:2 * _H])

    o = jnp.concatenate(outs, axis=0)
    h = h + _bdot(o, wo)
    m = jnp.maximum(_bdot(h, w1) + b1, 0.0)
    m = jnp.maximum(_bdot(m, w2) + b2, 0.0)
    h = h + m

    hd = _bdot(h, whead) + bhead
    gb = _BLOCK_ROWS // _L
    zmu_ref[...] = hd[:, 0:_T].reshape(gb, _L, _T)
    zsd_ref[...] = jnp.exp(hd[:, _T:2 * _T]).reshape(gb, _L, _T)
    smu_ref[...] = hd[:, 2 * _T:2 * _T + 1].reshape(gb, _L, 1)
    ssd_ref[...] = jnp.exp(hd[:, 2 * _T + 1:2 * _T + 2]).reshape(gb, _L, 1)


def _dense_stage(h_flat, wqkv, wo, w1, b1, w2, b2, whead, bhead, bias):
    n = h_flat.shape[0]
    grid = (n // _BLOCK_ROWS,)
    const = lambda shape: pl.BlockSpec(shape, lambda i: (0, 0))
    return pl.pallas_call(
        _dense_body,
        grid=grid,
        in_specs=[
            pl.BlockSpec((_BLOCK_ROWS, 2 * _H), lambda i: (i, 0)),
            const(wqkv.shape),
            const(wo.shape),
            const(w1.shape),
            const(b1.shape),
            const(w2.shape),
            const(b2.shape),
            const(whead.shape),
            const(bhead.shape),
            const(bias.shape),
        ],
        out_specs=[
            pl.BlockSpec((_BLOCK_ROWS // _L, _L, _T), lambda i: (i, 0, 0)),
            pl.BlockSpec((_BLOCK_ROWS // _L, _L, _T), lambda i: (i, 0, 0)),
            pl.BlockSpec((_BLOCK_ROWS // _L, _L, 1), lambda i: (i, 0, 0)),
            pl.BlockSpec((_BLOCK_ROWS // _L, _L, 1), lambda i: (i, 0, 0)),
        ],
        out_shape=[
            jax.ShapeDtypeStruct((n // _L, _L, _T), jnp.float32),
            jax.ShapeDtypeStruct((n // _L, _L, _T), jnp.float32),
            jax.ShapeDtypeStruct((n // _L, _L, 1), jnp.float32),
            jax.ShapeDtypeStruct((n // _L, _L, 1), jnp.float32),
        ],
        compiler_params=pltpu.CompilerParams(
            dimension_semantics=("parallel",),
        ),
    )(h_flat, wqkv, wo, w1, b1, w2, b2, whead, bhead, bias)


def _cross_batch_bias():
    ri = np.arange(_GR)[:, None] // _L
    cj = (np.arange(_NH * _GR)[None, :] % _GR) // _L
    return np.where(ri == cj, 0.0, -1e30).astype(np.float32)


def kernel(x, emb, attn_w, mlp_w, mlp_b, zmu_w, zmu_b, zlv_w, zlv_b,
           smu_w, smu_b, slv_w, slv_b):
    b, l = x.shape
    n = b * l
    emb_pad = jnp.pad(emb, ((0, 0), (0, 2 * _H - emb.shape[1])))
    idx2d = x.reshape(n // _WIN, _WIN).astype(jnp.int32)

    isq = 1.0 / np.sqrt(_DH)
    wqkv = jnp.concatenate(
        [attn_w[0, 0] * isq, attn_w[0, 1], attn_w[0, 2]],
        axis=1).astype(jnp.bfloat16)
    wo = attn_w[0, 3].astype(jnp.bfloat16)
    w1, w2 = mlp_w[0, 0].astype(jnp.bfloat16), mlp_w[0, 1].astype(jnp.bfloat16)
    b1, b2 = mlp_b[0, 0].reshape(1, _H), mlp_b[0, 1].reshape(1, _H)
    whead = jnp.concatenate(
        [zmu_w, 0.5 * zlv_w, smu_w, 0.5 * slv_w], axis=1).astype(jnp.bfloat16)
    bhead = jnp.concatenate(
        [zmu_b, 0.5 * zlv_b, smu_b, 0.5 * slv_b]).reshape(1, 2 * _T + 2)
    bias = jnp.asarray(_cross_batch_bias())

    h_flat = _sc_gather(emb_pad, idx2d)
    zmu, zsd, smu, ssd = _dense_stage(
        h_flat, wqkv, wo, w1, b1, w2, b2, whead, bhead, bias)
    return (zmu.reshape(b, l, _T), zsd.reshape(b, l, _T),
            smu.reshape(b, l, 1), ssd.reshape(b, l, 1))


# speedup vs baseline: 1.3264x; 1.0229x over previous
import functools

import jax
import jax.numpy as jnp
import numpy as np
from jax.experimental import pallas as pl
from jax.experimental.pallas import tpu as pltpu
from jax.experimental.pallas import tpu_sc as plsc

_V, _D, _H, _T = 1000000, 64, 64, 50
_NH = 4
_DH = _H // _NH
_L = 20

_GROUP_BATCH = 8
_GR = _GROUP_BATCH * _L
_GROUPS_PER_BLOCK = 32
_BLOCK_ROWS = _GR * _GROUPS_PER_BLOCK

_NW = 32
_WIN = 128
_RING = 4


def _sc_gather(emb_pad, idx2d):
    n_wins = idx2d.shape[0]
    d = emb_pad.shape[1]
    wins_per_worker = n_wins // _NW
    mesh = plsc.VectorSubcoreMesh(core_axis_name="c", subcore_axis_name="s")

    @functools.partial(
        pl.kernel,
        out_type=jax.ShapeDtypeStruct((n_wins * _WIN, d), emb_pad.dtype),
        mesh=mesh,
        scratch_types=[
            pltpu.VMEM((wins_per_worker, _WIN), jnp.int32),
            pltpu.VMEM((_RING * _WIN, d), emb_pad.dtype),
        ] + [pltpu.SemaphoreType.DMA] * _RING,
    )
    def gather_kernel(emb_hbm, i_hbm, o_hbm, idx_v, rows_v, *sems):
        wid = jax.lax.axis_index("s") * 2 + jax.lax.axis_index("c")
        win0 = wid * wins_per_worker

        pltpu.sync_copy(i_hbm.at[pl.ds(win0, wins_per_worker)], idx_v)

        def fire(slot, w):
            pltpu.async_copy(
                emb_hbm.at[idx_v.at[w]],
                rows_v.at[pl.ds(slot * _WIN, _WIN)],
                sems[slot],
            )

        def drain(slot):
            pltpu.make_async_copy(
                emb_hbm.at[idx_v.at[0]],
                rows_v.at[pl.ds(slot * _WIN, _WIN)],
                sems[slot],
            ).wait()

        for j in range(_RING):
            fire(j, j)

        @pl.loop(0, wins_per_worker // _RING)
        def _(c):
            for j in range(_RING):
                w = c * _RING + j
                drain(j)
                pltpu.sync_copy(
                    rows_v.at[pl.ds(j * _WIN, _WIN)],
                    o_hbm.at[pl.ds((win0 + w) * _WIN, _WIN)],
                )

                @pl.when(c < wins_per_worker // _RING - 1)
                def _():
                    fire(j, w + _RING)

    return gather_kernel(emb_pad, idx2d)


def _bdot(a, b):
    return jnp.dot(a.astype(jnp.bfloat16), b,
                   preferred_element_type=jnp.float32)


def _dense_body(h_ref, wqkv_ref, wo_ref, w1_ref, b1_ref, w2_ref, b2_ref,
                whead_ref, bhead_ref, bias_ref,
                zmu_ref, zsd_ref, smu_ref, ssd_ref):
    wqkv = wqkv_ref[...]
    wo = wo_ref[...]
    w1 = w1_ref[...]
    b1 = b1_ref[...]
    w2 = w2_ref[...]
    b2 = b2_ref[...]
    whead = whead_ref[...]
    bhead = bhead_ref[...]
    bias = bias_ref[...]

    h = h_ref[:, 0:_H]
    qkv = _bdot(h, wqkv)

    head_id = jax.lax.broadcasted_iota(jnp.int32, (_GR, _H), 1) // _DH
    zero = jnp.zeros((), jnp.bfloat16)
    m2 = jnp.concatenate(
        [(head_id == m).astype(jnp.bfloat16) for m in range(_NH)], axis=0)

    outs = []
    for g in range(_GROUPS_PER_BLOCK):
        r0 = g * _GR
        q = qkv[r0:r0 + _GR, 0:_H].astype(jnp.bfloat16)
        k = qkv[r0:r0 + _GR, _H:2 * _H].astype(jnp.bfloat16)
        v = qkv[r0:r0 + _GR, 2 * _H:3 * _H].astype(jnp.bfloat16)

        k2 = jnp.concatenate(
            [jnp.where(head_id == m, k, zero) for m in range(_NH)], axis=0)
        v2 = jnp.concatenate(
            [jnp.where(head_id == m, v, zero) for m in range(_NH)], axis=0)
        v3 = jnp.concatenate([v2, m2], axis=1)

        s = jax.lax.dot_general(
            q, k2, (((1,), (1,)), ((), ())),
            preferred_element_type=jnp.float32)
        p = jnp.exp(s + bias)

        c = _bdot(p, v3)
        outs.append(c[:, 0:_H] / c[:, _H:2 * _H])

    o = jnp.concatenate(outs, axis=0)
    h = h + _bdot(o, wo)
    m = jnp.maximum(_bdot(h, w1) + b1, 0.0)
    m = jnp.maximum(_bdot(m, w2) + b2, 0.0)
    h = h + m

    hd = _bdot(h, whead) + bhead
    gb = _BLOCK_ROWS // _L
    zmu_ref[...] = hd[:, 0:_T].reshape(gb, _L, _T)
    zsd_ref[...] = jnp.exp(hd[:, _T:2 * _T]).reshape(gb, _L, _T)
    smu_ref[...] = hd[:, 2 * _T:2 * _T + 1].reshape(gb, _L, 1)
    ssd_ref[...] = jnp.exp(hd[:, 2 * _T + 1:2 * _T + 2]).reshape(gb, _L, 1)


def _dense_stage(h_flat, wqkv, wo, w1, b1, w2, b2, whead, bhead, bias):
    n = h_flat.shape[0]
    grid = (n // _BLOCK_ROWS,)
    const = lambda shape: pl.BlockSpec(shape, lambda i: (0, 0))
    return pl.pallas_call(
        _dense_body,
        grid=grid,
        in_specs=[
            pl.BlockSpec((_BLOCK_ROWS, 2 * _H), lambda i: (i, 0)),
            const(wqkv.shape),
            const(wo.shape),
            const(w1.shape),
            const(b1.shape),
            const(w2.shape),
            const(b2.shape),
            const(whead.shape),
            const(bhead.shape),
            const(bias.shape),
        ],
        out_specs=[
            pl.BlockSpec((_BLOCK_ROWS // _L, _L, _T), lambda i: (i, 0, 0)),
            pl.BlockSpec((_BLOCK_ROWS // _L, _L, _T), lambda i: (i, 0, 0)),
            pl.BlockSpec((_BLOCK_ROWS // _L, _L, 1), lambda i: (i, 0, 0)),
            pl.BlockSpec((_BLOCK_ROWS // _L, _L, 1), lambda i: (i, 0, 0)),
        ],
        out_shape=[
            jax.ShapeDtypeStruct((n // _L, _L, _T), jnp.float32),
            jax.ShapeDtypeStruct((n // _L, _L, _T), jnp.float32),
            jax.ShapeDtypeStruct((n // _L, _L, 1), jnp.float32),
            jax.ShapeDtypeStruct((n // _L, _L, 1), jnp.float32),
        ],
        compiler_params=pltpu.CompilerParams(
            dimension_semantics=("parallel",),
        ),
    )(h_flat, wqkv, wo, w1, b1, w2, b2, whead, bhead, bias)


def _cross_batch_bias():
    ri = np.arange(_GR)[:, None] // _L
    cj = (np.arange(_NH * _GR)[None, :] % _GR) // _L
    return np.where(ri == cj, 0.0, -1e30).astype(np.float32)


def kernel(x, emb, attn_w, mlp_w, mlp_b, zmu_w, zmu_b, zlv_w, zlv_b,
           smu_w, smu_b, slv_w, slv_b):
    b, l = x.shape
    n = b * l
    emb_pad = jnp.pad(emb, ((0, 0), (0, 2 * _H - emb.shape[1])))
    idx2d = x.reshape(n // _WIN, _WIN).astype(jnp.int32)

    isq = 1.0 / np.sqrt(_DH)
    wqkv = jnp.concatenate(
        [attn_w[0, 0] * isq, attn_w[0, 1], attn_w[0, 2]],
        axis=1).astype(jnp.bfloat16)
    wo = attn_w[0, 3].astype(jnp.bfloat16)
    w1, w2 = mlp_w[0, 0].astype(jnp.bfloat16), mlp_w[0, 1].astype(jnp.bfloat16)
    b1, b2 = mlp_b[0, 0].reshape(1, _H), mlp_b[0, 1].reshape(1, _H)
    whead = jnp.concatenate(
        [zmu_w, 0.5 * zlv_w, smu_w, 0.5 * slv_w], axis=1).astype(jnp.bfloat16)
    bhead = jnp.concatenate(
        [zmu_b, 0.5 * zlv_b, smu_b, 0.5 * slv_b]).reshape(1, 2 * _T + 2)
    bias = jnp.asarray(_cross_batch_bias())

    h_flat = _sc_gather(emb_pad, idx2d)
    zmu, zsd, smu, ssd = _dense_stage(
        h_flat, wqkv, wo, w1, b1, w2, b2, whead, bhead, bias)
    return (zmu.reshape(b, l, _T), zsd.reshape(b, l, _T),
            smu.reshape(b, l, 1), ssd.reshape(b, l, 1))
